# baseline (speedup 1.0000x reference)
import numpy as np

import jax
import jax.numpy as jnp
from jax import lax
from jax.experimental import pallas as pl
from jax.experimental.pallas import tpu as pltpu
from jax.experimental.pallas import tpu_sc as plsc

_ALPHA = 0.5
_APPLY_PROB = 0.9
_K_NN = 15
_D = 1000
_NC = 2
_NS = 16


def _fold_draws():
    with jax.default_device(jax.local_devices(backend="cpu")[0]):
        key = jax.random.key(42)
        ks1, ks2, ka1, ka2 = jax.random.split(key, 4)
        gates = (jax.random.uniform(ks1, ()), jax.random.uniform(ks2, ()))
        out = []
        for ka, gate in zip((ka1, ka2), gates):
            kp, ko, kl, kb = jax.random.split(ka, 4)
            j = int(jax.random.choice(kp, jnp.arange(_K_NN, dtype=jnp.int32),
                                      shape=(1,), replace=False)[0])
            op = int(jax.random.randint(ko, (), 0, 3))
            lam = np.float32(
                (_ALPHA - 1.0) * np.float32(jax.random.uniform(kl, (1,), dtype=jnp.float32)[0])
                + np.float32(1.0))
            bern = np.asarray(
                jax.random.bernoulli(kb, _ALPHA, (_D,))).astype(np.float32)
            apply = bool(gate < _APPLY_PROB)
            geo = apply and op == 1
            if not apply:
                a = np.full((_D,), 1.0, np.float32)
                b = np.zeros((_D,), np.float32)
            elif op == 0:
                a = np.full((_D,), _ALPHA, np.float32)
                b = np.full((_D,), 1.0 - _ALPHA, np.float32)
            elif op == 1:
                a = np.full((_D,), lam, np.float32)
                b = np.full((_D,), 1.0 - lam, np.float32)
            else:
                a = bern
                b = 1.0 - bern
            out.append((j, a, b, geo))
        return out


_DRAWS = _fold_draws()
_GEO = (_DRAWS[0][3], _DRAWS[1][3])
_AB = np.stack([_DRAWS[0][1], _DRAWS[0][2], _DRAWS[1][1], _DRAWS[1][2]])
_JPOS = np.array([_DRAWS[0][0]] * 4 + [_DRAWS[1][0]] * 4, np.int32)


def _sc_pick_body(ipos_h, nn_h, out_h, idxv, nidv, sem):
    wid = lax.axis_index("s") * _NC + lax.axis_index("c")

    @pl.when(wid < 1)
    def _():
        pltpu.sync_copy(ipos_h, idxv)
        pltpu.async_copy(nn_h.at[idxv], nidv, sem).wait()
        pltpu.sync_copy(nidv, out_h)


def _tc_body(picks_s, x1_v, x2_v, ab_v, x_hbm, o1_v, o2_v, rows_v, sem0, sem1):
    c0 = pltpu.make_async_copy(x_hbm.at[pl.ds(picks_s[0], 1), :],
                               rows_v.at[pl.ds(0, 1), :], sem0)
    c1 = pltpu.make_async_copy(x_hbm.at[pl.ds(picks_s[4], 1), :],
                               rows_v.at[pl.ds(1, 1), :], sem1)
    c0.start()
    c1.start()
    c0.wait()
    c1.wait()
    for i, (x_v, o_v) in enumerate(((x1_v, o1_v), (x2_v, o2_v))):
        x = x_v[...]
        n = rows_v[pl.ds(i, 1), :]
        a = ab_v[pl.ds(2 * i, 1), :]
        b = ab_v[pl.ds(2 * i + 1, 1), :]
        if _GEO[i]:
            o_v[...] = jnp.exp(a * jnp.log(x) + b * jnp.log(n))
        else:
            o_v[...] = a * x + b * n


def kernel(x1, x2, cell_ids, X, nn_idx):
    cid = jnp.asarray(cell_ids, dtype=jnp.int32)
    ipos = cid * _K_NN + jnp.asarray(_JPOS)
    nn_flat = jnp.reshape(nn_idx, (-1,)).astype(jnp.int32)

    mesh = plsc.VectorSubcoreMesh(core_axis_name="c", subcore_axis_name="s",
                                  num_cores=_NC, num_subcores=_NS)
    picks = pl.kernel(
        _sc_pick_body,
        out_type=jax.ShapeDtypeStruct((8,), jnp.int32),
        mesh=mesh,
        scratch_types=[
            pltpu.VMEM((8,), jnp.int32),
            pltpu.VMEM((8,), jnp.int32),
            pltpu.SemaphoreType.DMA,
        ],
        compiler_params=pltpu.CompilerParams(needs_layout_passes=False),
        name="bbknn_nn_pick_sc",
    )(ipos, nn_flat)

    out1, out2 = pl.pallas_call(
        _tc_body,
        out_shape=(jax.ShapeDtypeStruct((1, _D), jnp.float32),
                   jax.ShapeDtypeStruct((1, _D), jnp.float32)),
        in_specs=[
            pl.BlockSpec(memory_space=pltpu.SMEM),
            pl.BlockSpec(memory_space=pltpu.VMEM),
            pl.BlockSpec(memory_space=pltpu.VMEM),
            pl.BlockSpec(memory_space=pltpu.VMEM),
            pl.BlockSpec(memory_space=pl.ANY),
        ],
        out_specs=(pl.BlockSpec(memory_space=pltpu.VMEM),
                   pl.BlockSpec(memory_space=pltpu.VMEM)),
        scratch_shapes=[
            pltpu.VMEM((2, _D), jnp.float32),
            pltpu.SemaphoreType.DMA,
            pltpu.SemaphoreType.DMA,
        ],
        name="bbknn_augment_tc",
    )(picks, jnp.reshape(x1, (1, _D)), jnp.reshape(x2, (1, _D)),
      jnp.asarray(_AB), X)

    return (out1, out2)


# speedup vs baseline: 6.9452x; 6.9452x over previous
import numpy as np

import jax
import jax.numpy as jnp
from jax import lax
from jax.experimental import pallas as pl
from jax.experimental.pallas import tpu as pltpu
from jax.experimental.pallas import tpu_sc as plsc

_ALPHA = 0.5
_APPLY_PROB = 0.9
_K_NN = 15
_N_CELLS = 50000
_D = 1000
_NC = 2
_NS = 16


def _fold_draws():
    with jax.default_device(jax.local_devices(backend="cpu")[0]):
        key = jax.random.key(42)
        ks1, ks2, ka1, ka2 = jax.random.split(key, 4)
        gates = (jax.random.uniform(ks1, ()), jax.random.uniform(ks2, ()))
        out = []
        for ka, gate in zip((ka1, ka2), gates):
            kp, ko, kl, kb = jax.random.split(ka, 4)
            j = int(jax.random.choice(kp, jnp.arange(_K_NN, dtype=jnp.int32),
                                      shape=(1,), replace=False)[0])
            op = int(jax.random.randint(ko, (), 0, 3))
            lam = np.float32(
                (_ALPHA - 1.0) * np.float32(jax.random.uniform(kl, (1,), dtype=jnp.float32)[0])
                + np.float32(1.0))
            bern = np.asarray(
                jax.random.bernoulli(kb, _ALPHA, (_D,))).astype(np.float32)
            apply = bool(gate < _APPLY_PROB)
            geo = apply and op == 1
            if not apply:
                a = np.full((_D,), 1.0, np.float32)
                b = np.zeros((_D,), np.float32)
            elif op == 0:
                a = np.full((_D,), _ALPHA, np.float32)
                b = np.full((_D,), 1.0 - _ALPHA, np.float32)
            elif op == 1:
                a = np.full((_D,), lam, np.float32)
                b = np.full((_D,), 1.0 - lam, np.float32)
            else:
                a = bern
                b = 1.0 - bern
            out.append((j, a, b, geo))
        return out


_DRAWS = _fold_draws()
_GEO = (_DRAWS[0][3], _DRAWS[1][3])
_AB = np.stack([_DRAWS[0][1], _DRAWS[0][2], _DRAWS[1][1], _DRAWS[1][2]], axis=1)
_JPOS = np.array([_DRAWS[0][0] * _N_CELLS] * 4 + [_DRAWS[1][0] * _N_CELLS] * 4,
                 np.int32)


def _sc_pick_body(ipos_h, nn_h, out_h, idxv, nidv, sem):
    wid = lax.axis_index("s") * _NC + lax.axis_index("c")

    @pl.when(wid < 1)
    def _():
        pltpu.sync_copy(ipos_h, idxv)
        pltpu.async_copy(nn_h.at[idxv], nidv, sem).wait()
        pltpu.sync_copy(nidv, out_h)


def _tc_body(picks_s, xs_v, ab_v, xt_hbm, out_v, slab0_v, slab1_v, sem0, sem1):
    p0 = picks_s[0]
    p1 = picks_s[4]
    c0 = pltpu.make_async_copy(
        xt_hbm.at[:, pl.ds(pl.multiple_of((p0 // 128) * 128, 128), 128)],
        slab0_v, sem0)
    c1 = pltpu.make_async_copy(
        xt_hbm.at[:, pl.ds(pl.multiple_of((p1 // 128) * 128, 128), 128)],
        slab1_v, sem1)
    c0.start()
    c1.start()
    c0.wait()
    c1.wait()
    lane = lax.broadcasted_iota(jnp.int32, (_D, 128), 1)
    for i, (slab, p) in enumerate(((slab0_v, p0), (slab1_v, p1))):
        onehot = (lane == p % 128).astype(jnp.float32)
        n = jnp.sum(slab[...] * onehot, axis=1, keepdims=True)
        x = xs_v[:, pl.ds(i, 1)]
        a = ab_v[:, pl.ds(2 * i, 1)]
        b = ab_v[:, pl.ds(2 * i + 1, 1)]
        if _GEO[i]:
            out_v[:, pl.ds(i, 1)] = jnp.exp(a * jnp.log(x) + b * jnp.log(n))
        else:
            out_v[:, pl.ds(i, 1)] = a * x + b * n


def kernel(x1, x2, cell_ids, X, nn_idx):
    cid = jnp.asarray(cell_ids, dtype=jnp.int32)
    ipos = cid + jnp.asarray(_JPOS)
    nn_flat = jnp.reshape(jnp.transpose(nn_idx), (-1,)).astype(jnp.int32)
    xt = jnp.transpose(X)
    xs = jnp.concatenate([jnp.reshape(x1, (_D, 1)), jnp.reshape(x2, (_D, 1))],
                         axis=1)

    mesh = plsc.VectorSubcoreMesh(core_axis_name="c", subcore_axis_name="s",
                                  num_cores=_NC, num_subcores=_NS)
    picks = pl.kernel(
        _sc_pick_body,
        out_type=jax.ShapeDtypeStruct((8,), jnp.int32),
        mesh=mesh,
        scratch_types=[
            pltpu.VMEM((8,), jnp.int32),
            pltpu.VMEM((8,), jnp.int32),
            pltpu.SemaphoreType.DMA,
        ],
        compiler_params=pltpu.CompilerParams(needs_layout_passes=False),
        name="bbknn_nn_pick_sc",
    )(ipos, nn_flat)

    out = pl.pallas_call(
        _tc_body,
        out_shape=jax.ShapeDtypeStruct((_D, 2), jnp.float32),
        in_specs=[
            pl.BlockSpec(memory_space=pltpu.SMEM),
            pl.BlockSpec(memory_space=pltpu.VMEM),
            pl.BlockSpec(memory_space=pltpu.VMEM),
            pl.BlockSpec(memory_space=pl.ANY),
        ],
        out_specs=pl.BlockSpec(memory_space=pltpu.VMEM),
        scratch_shapes=[
            pltpu.VMEM((_D, 128), jnp.float32),
            pltpu.VMEM((_D, 128), jnp.float32),
            pltpu.SemaphoreType.DMA,
            pltpu.SemaphoreType.DMA,
        ],
        name="bbknn_augment_tc",
    )(picks, xs, jnp.asarray(_AB), xt)

    outt = jnp.transpose(out)
    return (outt[0:1], outt[1:2])


# speedup vs baseline: 8.4149x; 1.2116x over previous
import numpy as np

import jax
import jax.numpy as jnp
from jax import lax
from jax.experimental import pallas as pl
from jax.experimental.pallas import tpu as pltpu
from jax.experimental.pallas import tpu_sc as plsc

_ALPHA = 0.5
_APPLY_PROB = 0.9
_K_NN = 15
_N_CELLS = 50000
_D = 1000
_NC = 2
_NS = 16


def _fold_draws():
    with jax.default_device(jax.local_devices(backend="cpu")[0]):
        key = jax.random.key(42)
        ks1, ks2, ka1, ka2 = jax.random.split(key, 4)
        gates = (jax.random.uniform(ks1, ()), jax.random.uniform(ks2, ()))
        out = []
        for ka, gate in zip((ka1, ka2), gates):
            kp, ko, kl, kb = jax.random.split(ka, 4)
            j = int(jax.random.choice(kp, jnp.arange(_K_NN, dtype=jnp.int32),
                                      shape=(1,), replace=False)[0])
            op = int(jax.random.randint(ko, (), 0, 3))
            lam = float(np.float32(
                (_ALPHA - 1.0) * np.float32(jax.random.uniform(kl, (1,), dtype=jnp.float32)[0])
                + np.float32(1.0)))
            bern = np.asarray(
                jax.random.bernoulli(kb, _ALPHA, (_D,))).astype(np.float32)
            apply = bool(gate < _APPLY_PROB)
            if not apply:
                mode, coef = "id", None
            elif op == 0:
                mode, coef = "lin", _ALPHA
            elif op == 1:
                mode, coef = "geo", lam
            else:
                mode, coef = "bern", bern.reshape(1, _D)
            out.append((j, mode, coef))
        return out


_DRAWS = _fold_draws()
_J = (_DRAWS[0][0], _DRAWS[1][0])


def _sc_pick_body(ipos_h, nn_h, out_h, idxv, nidv, sem):
    wid = lax.axis_index("s") * _NC + lax.axis_index("c")

    @pl.when(wid < 1)
    def _():
        pltpu.sync_copy(ipos_h, idxv)
        pltpu.async_copy(nn_h.at[idxv], nidv, sem).wait()
        pltpu.sync_copy(nidv, out_h)


def _augment(x, n, mode, coef):
    if mode == "id":
        return x
    if mode == "lin":
        return coef * x + (1.0 - coef) * n
    if mode == "geo":
        return jnp.exp(coef * jnp.log(x) + (1.0 - coef) * jnp.log(n))
    bern = jnp.asarray(coef)
    return bern * x + (1.0 - bern) * n


def _tc_body(picks_s, x1_v, x2_v, xt_hbm, o1_v, o2_v, slab0_v, slab1_v,
             sem0, sem1):
    p0 = picks_s[0]
    p1 = picks_s[4]
    c0 = pltpu.make_async_copy(
        xt_hbm.at[:, pl.ds(pl.multiple_of((p0 // 128) * 128, 128), 128)],
        slab0_v, sem0)
    c1 = pltpu.make_async_copy(
        xt_hbm.at[:, pl.ds(pl.multiple_of((p1 // 128) * 128, 128), 128)],
        slab1_v, sem1)
    c0.start()
    c1.start()
    c0.wait()
    c1.wait()
    lane = lax.broadcasted_iota(jnp.int32, (1, 128), 1)
    dn = (((1,), (1,)), ((), ()))
    for i, (slab, p, x_v, o_v) in enumerate(
            ((slab0_v, p0, x1_v, o1_v), (slab1_v, p1, x2_v, o2_v))):
        onehot = (lane == p % 128).astype(jnp.float32)
        n = lax.dot_general(onehot, slab[...], dn,
                            precision=lax.Precision.HIGHEST)
        o_v[...] = _augment(x_v[...], n, _DRAWS[i][1], _DRAWS[i][2])


def kernel(x1, x2, cell_ids, X, nn_idx):
    cid = jnp.asarray(cell_ids, dtype=jnp.int32)
    ipos = cid + jnp.asarray(np.array([0] * 4 + [_N_CELLS] * 4, np.int32))
    nnt = jnp.transpose(nn_idx)
    nn_flat = jnp.reshape(
        jnp.concatenate([nnt[_J[0]:_J[0] + 1], nnt[_J[1]:_J[1] + 1]], axis=0),
        (-1,)).astype(jnp.int32)
    xt = jnp.transpose(X)

    mesh = plsc.VectorSubcoreMesh(core_axis_name="c", subcore_axis_name="s",
                                  num_cores=_NC, num_subcores=_NS)
    picks = pl.kernel(
        _sc_pick_body,
        out_type=jax.ShapeDtypeStruct((8,), jnp.int32),
        mesh=mesh,
        scratch_types=[
            pltpu.VMEM((8,), jnp.int32),
            pltpu.VMEM((8,), jnp.int32),
            pltpu.SemaphoreType.DMA,
        ],
        compiler_params=pltpu.CompilerParams(needs_layout_passes=False),
        name="bbknn_nn_pick_sc",
    )(ipos, nn_flat)

    out1, out2 = pl.pallas_call(
        _tc_body,
        out_shape=(jax.ShapeDtypeStruct((1, _D), jnp.float32),
                   jax.ShapeDtypeStruct((1, _D), jnp.float32)),
        in_specs=[
            pl.BlockSpec(memory_space=pltpu.SMEM),
            pl.BlockSpec(memory_space=pltpu.VMEM),
            pl.BlockSpec(memory_space=pltpu.VMEM),
            pl.BlockSpec(memory_space=pl.ANY),
        ],
        out_specs=(pl.BlockSpec(memory_space=pltpu.VMEM),
                   pl.BlockSpec(memory_space=pltpu.VMEM)),
        scratch_shapes=[
            pltpu.VMEM((_D, 128), jnp.float32),
            pltpu.VMEM((_D, 128), jnp.float32),
            pltpu.SemaphoreType.DMA,
            pltpu.SemaphoreType.DMA,
        ],
        name="bbknn_augment_tc",
    )(picks, jnp.reshape(x1, (1, _D)), jnp.reshape(x2, (1, _D)), xt)

    return (out1, out2)


# speedup vs baseline: 8.5994x; 1.0219x over previous
import numpy as np

import jax
import jax.numpy as jnp
from jax import lax
from jax.experimental import pallas as pl
from jax.experimental.pallas import tpu as pltpu
from jax.experimental.pallas import tpu_sc as plsc

_ALPHA = 0.5
_APPLY_PROB = 0.9
_K_NN = 15
_N_CELLS = 50000
_D = 1000
_NC = 2
_NS = 16


def _fold_draws():
    with jax.default_device(jax.local_devices(backend="cpu")[0]):
        key = jax.random.key(42)
        ks1, ks2, ka1, ka2 = jax.random.split(key, 4)
        gates = (jax.random.uniform(ks1, ()), jax.random.uniform(ks2, ()))
        out = []
        for ka, gate in zip((ka1, ka2), gates):
            kp, ko, kl, kb = jax.random.split(ka, 4)
            j = int(jax.random.choice(kp, jnp.arange(_K_NN, dtype=jnp.int32),
                                      shape=(1,), replace=False)[0])
            op = int(jax.random.randint(ko, (), 0, 3))
            lam = float(np.float32(
                (_ALPHA - 1.0) * np.float32(jax.random.uniform(kl, (1,), dtype=jnp.float32)[0])
                + np.float32(1.0)))
            bern = np.asarray(
                jax.random.bernoulli(kb, _ALPHA, (_D,))).astype(np.float32)
            apply = bool(gate < _APPLY_PROB)
            if not apply:
                mode, coef = "id", None
            elif op == 0:
                mode, coef = "lin", _ALPHA
            elif op == 1:
                mode, coef = "geo", lam
            else:
                mode, coef = "bern", bern.reshape(1, _D)
            out.append((j, mode, coef))
        return out


_DRAWS = _fold_draws()
_J = (_DRAWS[0][0], _DRAWS[1][0])


def _sc_pick_body(ipos_h, nn_h, out_h, idxv, nidv, sem):
    wid = lax.axis_index("s") * _NC + lax.axis_index("c")

    @pl.when(wid < 1)
    def _():
        pltpu.sync_copy(ipos_h, idxv)
        pltpu.async_copy(nn_h.at[idxv], nidv, sem).wait()
        pltpu.sync_copy(nidv, out_h)


def _augment(x, n, mode, coef):
    if mode == "id":
        return x
    if mode == "lin":
        return coef * x + (1.0 - coef) * n
    if mode == "geo":
        return jnp.exp(coef * jnp.log(x) + (1.0 - coef) * jnp.log(n))
    bern = jnp.asarray(coef)
    return bern * x + (1.0 - bern) * n


def _tc_body(picks_s, x1_v, x2_v, xt_hbm, o1_v, o2_v, slab0_v, slab1_v,
             sem0, sem1):
    p0 = picks_s[0]
    p1 = picks_s[4]
    c0 = pltpu.make_async_copy(
        xt_hbm.at[:, pl.ds(pl.multiple_of((p0 // 128) * 128, 128), 128)],
        slab0_v, sem0)
    c1 = pltpu.make_async_copy(
        xt_hbm.at[:, pl.ds(pl.multiple_of((p1 // 128) * 128, 128), 128)],
        slab1_v, sem1)
    c0.start()
    c1.start()
    c0.wait()
    c1.wait()
    lane = lax.broadcasted_iota(jnp.int32, (1, 128), 1)
    dn = (((1,), (1,)), ((), ()))
    for i, (slab, p, x_v, o_v) in enumerate(
            ((slab0_v, p0, x1_v, o1_v), (slab1_v, p1, x2_v, o2_v))):
        onehot = (lane == p % 128).astype(jnp.float32)
        n = lax.dot_general(onehot, slab[...], dn,
                            precision=lax.Precision.HIGHEST)
        o_v[...] = _augment(x_v[...], n, _DRAWS[i][1], _DRAWS[i][2])


def kernel(x1, x2, cell_ids, X, nn_idx):
    lo, hi = min(_J), max(_J)
    cid = jnp.asarray(cell_ids, dtype=jnp.int32)
    ipos = cid + jnp.asarray(np.array(
        [(_J[0] - lo) * _N_CELLS] * 4 + [(_J[1] - lo) * _N_CELLS] * 4,
        np.int32))
    nnt = jnp.transpose(nn_idx)
    nn_flat = jnp.reshape(nnt[lo:hi + 1], (-1,)).astype(jnp.int32)
    xt = jnp.transpose(X)

    mesh = plsc.VectorSubcoreMesh(core_axis_name="c", subcore_axis_name="s",
                                  num_cores=_NC, num_subcores=_NS)
    picks = pl.kernel(
        _sc_pick_body,
        out_type=jax.ShapeDtypeStruct((8,), jnp.int32),
        mesh=mesh,
        scratch_types=[
            pltpu.VMEM((8,), jnp.int32),
            pltpu.VMEM((8,), jnp.int32),
            pltpu.SemaphoreType.DMA,
        ],
        compiler_params=pltpu.CompilerParams(needs_layout_passes=False),
        name="bbknn_nn_pick_sc",
    )(ipos, nn_flat)

    out1, out2 = pl.pallas_call(
        _tc_body,
        out_shape=(jax.ShapeDtypeStruct((1, _D), jnp.float32),
                   jax.ShapeDtypeStruct((1, _D), jnp.float32)),
        in_specs=[
            pl.BlockSpec(memory_space=pltpu.SMEM),
            pl.BlockSpec(memory_space=pltpu.VMEM),
            pl.BlockSpec(memory_space=pltpu.VMEM),
            pl.BlockSpec(memory_space=pl.ANY),
        ],
        out_specs=(pl.BlockSpec(memory_space=pltpu.VMEM),
                   pl.BlockSpec(memory_space=pltpu.VMEM)),
        scratch_shapes=[
            pltpu.VMEM((_D, 128), jnp.float32),
            pltpu.VMEM((_D, 128), jnp.float32),
            pltpu.SemaphoreType.DMA,
            pltpu.SemaphoreType.DMA,
        ],
        name="bbknn_augment_tc",
    )(picks, jnp.reshape(x1, (1, _D)), jnp.reshape(x2, (1, _D)), xt)

    return (out1, out2)


# speedup vs baseline: 8.6197x; 1.0024x over previous
import numpy as np

import jax
import jax.numpy as jnp
from jax import lax
from jax.experimental import pallas as pl
from jax.experimental.pallas import tpu as pltpu
from jax.experimental.pallas import tpu_sc as plsc

_ALPHA = 0.5
_APPLY_PROB = 0.9
_K_NN = 15
_N_CELLS = 50000
_D = 1000
_NC = 2
_NS = 16


def _fold_draws():
    with jax.default_device(jax.local_devices(backend="cpu")[0]):
        key = jax.random.key(42)
        ks1, ks2, ka1, ka2 = jax.random.split(key, 4)
        gates = (jax.random.uniform(ks1, ()), jax.random.uniform(ks2, ()))
        out = []
        for ka, gate in zip((ka1, ka2), gates):
            kp, ko, kl, kb = jax.random.split(ka, 4)
            j = int(jax.random.choice(kp, jnp.arange(_K_NN, dtype=jnp.int32),
                                      shape=(1,), replace=False)[0])
            op = int(jax.random.randint(ko, (), 0, 3))
            lam = float(np.float32(
                (_ALPHA - 1.0) * np.float32(jax.random.uniform(kl, (1,), dtype=jnp.float32)[0])
                + np.float32(1.0)))
            bern = np.asarray(
                jax.random.bernoulli(kb, _ALPHA, (_D,))).astype(np.float32)
            apply = bool(gate < _APPLY_PROB)
            if not apply:
                mode, coef = "id", None
            elif op == 0:
                mode, coef = "lin", _ALPHA
            elif op == 1:
                mode, coef = "geo", lam
            else:
                mode, coef = "bern", bern.reshape(1, _D)
            out.append((j, mode, coef))
        return out


_DRAWS = _fold_draws()
_J = (_DRAWS[0][0], _DRAWS[1][0])


def _sc_pick_body(ipos_h, nn_h, out_h, idxv, nidv, sem):
    wid = lax.axis_index("s") * _NC + lax.axis_index("c")

    @pl.when(wid < 1)
    def _():
        pltpu.sync_copy(ipos_h, idxv)
        pltpu.async_copy(nn_h.at[idxv], nidv, sem).wait()
        pltpu.sync_copy(nidv, out_h)


def _augment(x, n, mode, coef):
    if mode == "id":
        return x
    if mode == "lin":
        return coef * x + (1.0 - coef) * n
    if mode == "geo":
        return jnp.exp(coef * jnp.log(x) + (1.0 - coef) * jnp.log(n))
    bern = jnp.asarray(coef)
    return bern * x + (1.0 - bern) * n


def _tc_body(picks_s, x1_v, x2_v, xt_hbm, o1_v, o2_v, slab0_v, slab1_v,
             sem0, sem1):
    p0 = picks_s[0]
    p1 = picks_s[4]
    c0 = pltpu.make_async_copy(
        xt_hbm.at[:, pl.ds(pl.multiple_of((p0 // 128) * 128, 128), 128)],
        slab0_v, sem0)
    c1 = pltpu.make_async_copy(
        xt_hbm.at[:, pl.ds(pl.multiple_of((p1 // 128) * 128, 128), 128)],
        slab1_v, sem1)
    c0.start()
    c1.start()
    lane = lax.broadcasted_iota(jnp.int32, (1, 128), 1)
    dn = (((1,), (1,)), ((), ()))
    for i, (slab, p, x_v, o_v, c) in enumerate(
            ((slab0_v, p0, x1_v, o1_v, c0), (slab1_v, p1, x2_v, o2_v, c1))):
        c.wait()
        onehot = (lane == p % 128).astype(jnp.float32)
        n = lax.dot_general(onehot, slab[...], dn,
                            precision=lax.Precision.HIGHEST)
        o_v[...] = _augment(x_v[...], n, _DRAWS[i][1], _DRAWS[i][2])


def kernel(x1, x2, cell_ids, X, nn_idx):
    lo, hi = min(_J), max(_J)
    cid = jnp.asarray(cell_ids, dtype=jnp.int32)
    ipos = cid + jnp.asarray(np.array(
        [(_J[0] - lo) * _N_CELLS] * 4 + [(_J[1] - lo) * _N_CELLS] * 4,
        np.int32))
    nnt = jnp.transpose(nn_idx)
    nn_flat = jnp.reshape(nnt[lo:hi + 1], (-1,)).astype(jnp.int32)
    xt = jnp.transpose(X)

    mesh = plsc.VectorSubcoreMesh(core_axis_name="c", subcore_axis_name="s",
                                  num_cores=_NC, num_subcores=_NS)
    picks = pl.kernel(
        _sc_pick_body,
        out_type=jax.ShapeDtypeStruct((8,), jnp.int32),
        mesh=mesh,
        scratch_types=[
            pltpu.VMEM((8,), jnp.int32),
            pltpu.VMEM((8,), jnp.int32),
            pltpu.SemaphoreType.DMA,
        ],
        compiler_params=pltpu.CompilerParams(needs_layout_passes=False),
        name="bbknn_nn_pick_sc",
    )(ipos, nn_flat)

    out1, out2 = pl.pallas_call(
        _tc_body,
        out_shape=(jax.ShapeDtypeStruct((1, _D), jnp.float32),
                   jax.ShapeDtypeStruct((1, _D), jnp.float32)),
        in_specs=[
            pl.BlockSpec(memory_space=pltpu.SMEM),
            pl.BlockSpec(memory_space=pltpu.VMEM),
            pl.BlockSpec(memory_space=pltpu.VMEM),
            pl.BlockSpec(memory_space=pl.ANY),
        ],
        out_specs=(pl.BlockSpec(memory_space=pltpu.VMEM),
                   pl.BlockSpec(memory_space=pltpu.VMEM)),
        scratch_shapes=[
            pltpu.VMEM((_D, 128), jnp.float32),
            pltpu.VMEM((_D, 128), jnp.float32),
            pltpu.SemaphoreType.DMA,
            pltpu.SemaphoreType.DMA,
        ],
        name="bbknn_augment_tc",
    )(picks, jnp.reshape(x1, (1, _D)), jnp.reshape(x2, (1, _D)), xt)

    return (out1, out2)


# speedup vs baseline: 8.9944x; 1.0435x over previous
import numpy as np

import jax
import jax.numpy as jnp
from jax import lax
from jax.experimental import pallas as pl
from jax.experimental.pallas import tpu as pltpu
from jax.experimental.pallas import tpu_sc as plsc

_ALPHA = 0.5
_APPLY_PROB = 0.9
_K_NN = 15
_N_CELLS = 50000
_D = 1000
_NC = 2
_NS = 16


def _fold_draws():
    with jax.default_device(jax.local_devices(backend="cpu")[0]):
        key = jax.random.key(42)
        ks1, ks2, ka1, ka2 = jax.random.split(key, 4)
        gates = (jax.random.uniform(ks1, ()), jax.random.uniform(ks2, ()))
        out = []
        for ka, gate in zip((ka1, ka2), gates):
            kp, ko, kl, kb = jax.random.split(ka, 4)
            j = int(jax.random.choice(kp, jnp.arange(_K_NN, dtype=jnp.int32),
                                      shape=(1,), replace=False)[0])
            op = int(jax.random.randint(ko, (), 0, 3))
            lam = float(np.float32(
                (_ALPHA - 1.0) * np.float32(jax.random.uniform(kl, (1,), dtype=jnp.float32)[0])
                + np.float32(1.0)))
            bern = np.asarray(
                jax.random.bernoulli(kb, _ALPHA, (_D,))).astype(np.float32)
            apply = bool(gate < _APPLY_PROB)
            if not apply:
                mode, coef = "id", None
            elif op == 0:
                mode, coef = "lin", _ALPHA
            elif op == 1:
                mode, coef = "geo", lam
            else:
                mode, coef = "bern", bern.reshape(1, _D)
            out.append((j, mode, coef))
        return out


_DRAWS = _fold_draws()
_J = (_DRAWS[0][0], _DRAWS[1][0])


def _sc_pick_body(ipos_h, nn_h, out_h, idxv, nidv, sem):
    wid = lax.axis_index("s") * _NC + lax.axis_index("c")

    @pl.when(wid < 1)
    def _():
        pltpu.sync_copy(ipos_h, idxv)
        pltpu.async_copy(nn_h.at[idxv], nidv, sem).wait()
        pltpu.sync_copy(nidv, out_h)


def _augment(x, n, mode, coef):
    if mode == "id":
        return x
    if mode == "lin":
        return coef * x + (1.0 - coef) * n
    if mode == "geo":
        return jnp.exp(coef * jnp.log(x) + (1.0 - coef) * jnp.log(n))
    bern = jnp.asarray(coef)
    return bern * x + (1.0 - bern) * n


def _tc_body(picks_s, x1_v, x2_v, xt_hbm, o1_v, o2_v, slab0_v, slab1_v,
             sem0, sem1):
    p0 = picks_s[0]
    p1 = picks_s[4]
    c0 = pltpu.make_async_copy(
        xt_hbm.at[:, pl.ds(pl.multiple_of((p0 // 128) * 128, 128), 128)],
        slab0_v, sem0)
    c1 = pltpu.make_async_copy(
        xt_hbm.at[:, pl.ds(pl.multiple_of((p1 // 128) * 128, 128), 128)],
        slab1_v, sem1)
    c0.start()
    c1.start()
    lane = lax.broadcasted_iota(jnp.int32, (1, 128), 1)
    dn = (((1,), (1,)), ((), ()))
    for i, (slab, p, x_v, o_v, c) in enumerate(
            ((slab0_v, p0, x1_v, o1_v, c0), (slab1_v, p1, x2_v, o2_v, c1))):
        c.wait()
        onehot = (lane == p % 128).astype(jnp.float32)
        n = lax.dot_general(onehot, slab[...], dn,
                            precision=lax.Precision.HIGHEST)
        o_v[...] = _augment(x_v[...], n, _DRAWS[i][1], _DRAWS[i][2])


def kernel(x1, x2, cell_ids, X, nn_idx):
    lo, hi = min(_J), max(_J)
    cid = jnp.asarray(cell_ids, dtype=jnp.int32)
    ipos = cid + jnp.asarray(np.array(
        [(_J[0] - lo) * _N_CELLS] * 4 + [(_J[1] - lo) * _N_CELLS] * 4,
        np.int32))
    nnt = jnp.transpose(nn_idx)
    nn_flat = jnp.reshape(nnt[lo:hi + 1], (-1,)).astype(jnp.int32)
    xt = jnp.transpose(X)

    mesh = plsc.VectorSubcoreMesh(core_axis_name="c", subcore_axis_name="s",
                                  num_cores=1, num_subcores=_NS)
    picks = pl.kernel(
        _sc_pick_body,
        out_type=jax.ShapeDtypeStruct((8,), jnp.int32),
        mesh=mesh,
        scratch_types=[
            pltpu.VMEM((8,), jnp.int32),
            pltpu.VMEM((8,), jnp.int32),
            pltpu.SemaphoreType.DMA,
        ],
        compiler_params=pltpu.CompilerParams(needs_layout_passes=False),
        name="bbknn_nn_pick_sc",
    )(ipos, nn_flat)

    out1, out2 = pl.pallas_call(
        _tc_body,
        out_shape=(jax.ShapeDtypeStruct((1, _D), jnp.float32),
                   jax.ShapeDtypeStruct((1, _D), jnp.float32)),
        in_specs=[
            pl.BlockSpec(memory_space=pltpu.SMEM),
            pl.BlockSpec(memory_space=pltpu.VMEM),
            pl.BlockSpec(memory_space=pltpu.VMEM),
            pl.BlockSpec(memory_space=pl.ANY),
        ],
        out_specs=(pl.BlockSpec(memory_space=pltpu.VMEM),
                   pl.BlockSpec(memory_space=pltpu.VMEM)),
        scratch_shapes=[
            pltpu.VMEM((_D, 128), jnp.float32),
            pltpu.VMEM((_D, 128), jnp.float32),
            pltpu.SemaphoreType.DMA,
            pltpu.SemaphoreType.DMA,
        ],
        name="bbknn_augment_tc",
    )(picks, jnp.reshape(x1, (1, _D)), jnp.reshape(x2, (1, _D)), xt)

    return (out1, out2)


# speedup vs baseline: 9.1938x; 1.0222x over previous
import numpy as np

import jax
import jax.numpy as jnp
from jax import lax
from jax.experimental import pallas as pl
from jax.experimental.pallas import tpu as pltpu
from jax.experimental.pallas import tpu_sc as plsc

_ALPHA = 0.5
_APPLY_PROB = 0.9
_K_NN = 15
_N_CELLS = 50000
_D = 1000
_NC = 2
_NS = 16


def _fold_draws():
    with jax.default_device(jax.local_devices(backend="cpu")[0]):
        key = jax.random.key(42)
        ks1, ks2, ka1, ka2 = jax.random.split(key, 4)
        gates = (jax.random.uniform(ks1, ()), jax.random.uniform(ks2, ()))
        out = []
        for ka, gate in zip((ka1, ka2), gates):
            kp, ko, kl, kb = jax.random.split(ka, 4)
            j = int(jax.random.choice(kp, jnp.arange(_K_NN, dtype=jnp.int32),
                                      shape=(1,), replace=False)[0])
            op = int(jax.random.randint(ko, (), 0, 3))
            lam = float(np.float32(
                (_ALPHA - 1.0) * np.float32(jax.random.uniform(kl, (1,), dtype=jnp.float32)[0])
                + np.float32(1.0)))
            bern = np.asarray(
                jax.random.bernoulli(kb, _ALPHA, (_D,))).astype(np.float32)
            apply = bool(gate < _APPLY_PROB)
            if not apply:
                mode, coef = "id", None
            elif op == 0:
                mode, coef = "lin", _ALPHA
            elif op == 1:
                mode, coef = "geo", lam
            else:
                mode, coef = "bern", bern.reshape(1, _D)
            out.append((j, mode, coef))
        return out


_DRAWS = _fold_draws()
_J = (_DRAWS[0][0], _DRAWS[1][0])


def _sc_pick_body(ipos_h, nn_h, out_h, idxv, nidv, sem):
    wid = lax.axis_index("s") * _NC + lax.axis_index("c")

    @pl.when(wid < 1)
    def _():
        pltpu.sync_copy(ipos_h, idxv)
        pltpu.async_copy(nn_h.at[idxv], nidv, sem).wait()
        pltpu.sync_copy(nidv, out_h)


def _augment(x, n, mode, coef):
    if mode == "id":
        return x
    if mode == "lin":
        return coef * x + (1.0 - coef) * n
    if mode == "geo":
        return jnp.exp(coef * jnp.log(x) + (1.0 - coef) * jnp.log(n))
    bern = jnp.asarray(coef)
    return bern * x + (1.0 - bern) * n


def _tc_body(picks_s, x1_v, x2_v, xt_hbm, o1_v, o2_v, slab0_v, slab1_v,
             sem0, sem1):
    p0 = picks_s[0]
    p1 = picks_s[4]
    c0 = pltpu.make_async_copy(
        xt_hbm.at[:, pl.ds(pl.multiple_of((p0 // 128) * 128, 128), 128)],
        slab0_v, sem0)
    c1 = pltpu.make_async_copy(
        xt_hbm.at[:, pl.ds(pl.multiple_of((p1 // 128) * 128, 128), 128)],
        slab1_v, sem1)
    c0.start()
    c1.start()
    lane = lax.broadcasted_iota(jnp.int32, (1, 128), 1)
    dn = (((1,), (1,)), ((), ()))
    for i, (slab, p, x_v, o_v, c) in enumerate(
            ((slab0_v, p0, x1_v, o1_v, c0), (slab1_v, p1, x2_v, o2_v, c1))):
        c.wait()
        onehot = (lane == p % 128).astype(jnp.float32)
        n = lax.dot_general(onehot, slab[...], dn,
                            precision=lax.Precision.HIGHEST)
        o_v[...] = _augment(x_v[...], n, _DRAWS[i][1], _DRAWS[i][2])


def kernel(x1, x2, cell_ids, X, nn_idx):
    lo, hi = min(_J), max(_J)
    cid = jnp.asarray(cell_ids, dtype=jnp.int32)
    ipos = cid + jnp.asarray(np.array(
        [(_J[0] - lo) * _N_CELLS] * 4 + [(_J[1] - lo) * _N_CELLS] * 4,
        np.int32))
    nnt = jnp.transpose(nn_idx)
    nn_flat = jnp.reshape(nnt[lo:hi + 1], (-1,)).astype(jnp.int32)
    xt = jnp.transpose(X)

    mesh = plsc.VectorSubcoreMesh(core_axis_name="c", subcore_axis_name="s",
                                  num_cores=1, num_subcores=1)
    picks = pl.kernel(
        _sc_pick_body,
        out_type=jax.ShapeDtypeStruct((8,), jnp.int32),
        mesh=mesh,
        scratch_types=[
            pltpu.VMEM((8,), jnp.int32),
            pltpu.VMEM((8,), jnp.int32),
            pltpu.SemaphoreType.DMA,
        ],
        compiler_params=pltpu.CompilerParams(needs_layout_passes=False),
        name="bbknn_nn_pick_sc",
    )(ipos, nn_flat)

    out1, out2 = pl.pallas_call(
        _tc_body,
        out_shape=(jax.ShapeDtypeStruct((1, _D), jnp.float32),
                   jax.ShapeDtypeStruct((1, _D), jnp.float32)),
        in_specs=[
            pl.BlockSpec(memory_space=pltpu.SMEM),
            pl.BlockSpec(memory_space=pltpu.VMEM),
            pl.BlockSpec(memory_space=pltpu.VMEM),
            pl.BlockSpec(memory_space=pl.ANY),
        ],
        out_specs=(pl.BlockSpec(memory_space=pltpu.VMEM),
                   pl.BlockSpec(memory_space=pltpu.VMEM)),
        scratch_shapes=[
            pltpu.VMEM((_D, 128), jnp.float32),
            pltpu.VMEM((_D, 128), jnp.float32),
            pltpu.SemaphoreType.DMA,
            pltpu.SemaphoreType.DMA,
        ],
        name="bbknn_augment_tc",
    )(picks, jnp.reshape(x1, (1, _D)), jnp.reshape(x2, (1, _D)), xt)

    return (out1, out2)


# speedup vs baseline: 9.2205x; 1.0029x over previous
import numpy as np

import jax
import jax.numpy as jnp
from jax import lax
from jax.experimental import pallas as pl
from jax.experimental.pallas import tpu as pltpu
from jax.experimental.pallas import tpu_sc as plsc

_ALPHA = 0.5
_APPLY_PROB = 0.9
_K_NN = 15
_N_CELLS = 50000
_D = 1000
_NC = 2
_NS = 16


def _fold_draws():
    with jax.default_device(jax.local_devices(backend="cpu")[0]):
        key = jax.random.key(42)
        ks1, ks2, ka1, ka2 = jax.random.split(key, 4)
        gates = (jax.random.uniform(ks1, ()), jax.random.uniform(ks2, ()))
        out = []
        for ka, gate in zip((ka1, ka2), gates):
            kp, ko, kl, kb = jax.random.split(ka, 4)
            j = int(jax.random.choice(kp, jnp.arange(_K_NN, dtype=jnp.int32),
                                      shape=(1,), replace=False)[0])
            op = int(jax.random.randint(ko, (), 0, 3))
            lam = float(np.float32(
                (_ALPHA - 1.0) * np.float32(jax.random.uniform(kl, (1,), dtype=jnp.float32)[0])
                + np.float32(1.0)))
            bern = np.asarray(
                jax.random.bernoulli(kb, _ALPHA, (_D,))).astype(np.float32)
            apply = bool(gate < _APPLY_PROB)
            if not apply:
                mode, coef = "id", None
            elif op == 0:
                mode, coef = "lin", _ALPHA
            elif op == 1:
                mode, coef = "geo", lam
            else:
                mode, coef = "bern", bern.reshape(1, _D)
            out.append((j, mode, coef))
        return out


_DRAWS = _fold_draws()
_J = (_DRAWS[0][0], _DRAWS[1][0])


def _sc_pick_body(ipos_h, nn_h, out_h, idxv, nidv, sem):
    pltpu.sync_copy(ipos_h, idxv)
    pltpu.async_copy(nn_h.at[idxv], nidv, sem).wait()
    pltpu.sync_copy(nidv, out_h)


def _augment(x, n, mode, coef):
    if mode == "id":
        return x
    if mode == "lin":
        return coef * x + (1.0 - coef) * n
    if mode == "geo":
        return jnp.exp(coef * jnp.log(x) + (1.0 - coef) * jnp.log(n))
    bern = jnp.asarray(coef)
    return bern * x + (1.0 - bern) * n


def _tc_body(picks_s, x1_v, x2_v, xt_hbm, o1_v, o2_v, slab0_v, slab1_v,
             sem0, sem1):
    p0 = picks_s[0]
    p1 = picks_s[4]
    c0 = pltpu.make_async_copy(
        xt_hbm.at[:, pl.ds(pl.multiple_of((p0 // 128) * 128, 128), 128)],
        slab0_v, sem0)
    c1 = pltpu.make_async_copy(
        xt_hbm.at[:, pl.ds(pl.multiple_of((p1 // 128) * 128, 128), 128)],
        slab1_v, sem1)
    c0.start()
    c1.start()
    lane = lax.broadcasted_iota(jnp.int32, (1, 128), 1)
    dn = (((1,), (1,)), ((), ()))
    for i, (slab, p, x_v, o_v, c) in enumerate(
            ((slab0_v, p0, x1_v, o1_v, c0), (slab1_v, p1, x2_v, o2_v, c1))):
        c.wait()
        onehot = (lane == p % 128).astype(jnp.float32)
        n = lax.dot_general(onehot, slab[...], dn,
                            precision=lax.Precision.HIGHEST)
        o_v[...] = _augment(x_v[...], n, _DRAWS[i][1], _DRAWS[i][2])


def kernel(x1, x2, cell_ids, X, nn_idx):
    lo, hi = min(_J), max(_J)
    cid = jnp.asarray(cell_ids, dtype=jnp.int32)
    ipos = cid + jnp.asarray(np.array(
        [(_J[0] - lo) * _N_CELLS] * 4 + [(_J[1] - lo) * _N_CELLS] * 4,
        np.int32))
    nnt = jnp.transpose(nn_idx)
    nn_flat = jnp.reshape(nnt[lo:hi + 1], (-1,)).astype(jnp.int32)
    xt = jnp.transpose(X)

    mesh = plsc.VectorSubcoreMesh(core_axis_name="c", subcore_axis_name="s",
                                  num_cores=1, num_subcores=1)
    picks = pl.kernel(
        _sc_pick_body,
        out_type=jax.ShapeDtypeStruct((8,), jnp.int32),
        mesh=mesh,
        scratch_types=[
            pltpu.VMEM((8,), jnp.int32),
            pltpu.VMEM((8,), jnp.int32),
            pltpu.SemaphoreType.DMA,
        ],
        compiler_params=pltpu.CompilerParams(needs_layout_passes=False),
        name="bbknn_nn_pick_sc",
    )(ipos, nn_flat)

    out1, out2 = pl.pallas_call(
        _tc_body,
        out_shape=(jax.ShapeDtypeStruct((1, _D), jnp.float32),
                   jax.ShapeDtypeStruct((1, _D), jnp.float32)),
        in_specs=[
            pl.BlockSpec(memory_space=pltpu.SMEM),
            pl.BlockSpec(memory_space=pltpu.VMEM),
            pl.BlockSpec(memory_space=pltpu.VMEM),
            pl.BlockSpec(memory_space=pl.ANY),
        ],
        out_specs=(pl.BlockSpec(memory_space=pltpu.VMEM),
                   pl.BlockSpec(memory_space=pltpu.VMEM)),
        scratch_shapes=[
            pltpu.VMEM((_D, 128), jnp.float32),
            pltpu.VMEM((_D, 128), jnp.float32),
            pltpu.SemaphoreType.DMA,
            pltpu.SemaphoreType.DMA,
        ],
        name="bbknn_augment_tc",
    )(picks, jnp.reshape(x1, (1, _D)), jnp.reshape(x2, (1, _D)), xt)

    return (out1, out2)
